# baseline (speedup 1.0000x reference)
import functools

import jax
import jax.numpy as jnp
from jax import lax
from jax.experimental import pallas as pl
from jax.experimental.pallas import tpu as pltpu
from jax.experimental.pallas import tpu_sc as plsc

_M = 7
_TAU = 3


def kernel(x):
    B, S, C = x.shape
    nwin = _M + 1
    halo = _M * _TAU
    valid_t = S - halo
    nblk = 16
    tblk = 128
    tail = valid_t - (nblk - 1) * tblk
    tch = 32
    nch = tblk // tch
    glen = tch + ((halo + 7) // 8) * 8
    ntail = ((tail - (nch - 1) * tch + 7) // 8) * 8
    ngrp = C // 16

    mesh = plsc.VectorSubcoreMesh(core_axis_name="c", subcore_axis_name="s")

    @functools.partial(
        pl.kernel,
        mesh=mesh,
        out_type=jax.ShapeDtypeStruct((B, valid_t, nwin * C), jnp.float32),
        scratch_types=[
            pltpu.VMEM((glen, C), jnp.float32),
            pltpu.VMEM((2, tch, C), jnp.float32),
            pltpu.SemaphoreType.DMA,
            pltpu.SemaphoreType.DMA,
        ],
        compiler_params=pltpu.CompilerParams(disable_bounds_checks=True),
    )
    def run(x_hbm, out_hbm, buf_in, buf_st, gsem, ssem):
        cc = lax.axis_index("c")
        ss = lax.axis_index("s")
        wid = ss * 2 + cc
        b = wid // nblk
        j = wid % nblk

        def wait_scatter(rows):
            pltpu.make_async_copy(
                buf_st.at[0, pl.ds(0, rows), :],
                out_hbm.at[b, pl.ds(0, rows), pl.ds(0, C)],
                ssem,
            ).wait()

        def chunk_body(k, carry):
            t0 = pl.multiple_of(j * tblk + k * tch, 8)
            is_tail = jnp.logical_and(j == nblk - 1, k == nch - 1)

            @pl.when(jnp.logical_not(is_tail))
            def _gather_full():
                pltpu.async_copy(
                    x_hbm.at[b, pl.ds(t0, glen), :], buf_in, gsem
                ).wait()

            @pl.when(is_tail)
            def _gather_tail():
                pltpu.async_copy(
                    x_hbm.at[b, pl.ds(t0, tch), :],
                    buf_in.at[pl.ds(0, tch)],
                    gsem,
                ).wait()

            def win_body(i, carry2):
                slot = i % 2
                do_wait = jnp.logical_or(k > 0, i >= 2)
                prev_tail = jnp.logical_and(is_tail, i >= 2)

                @pl.when(jnp.logical_and(do_wait, jnp.logical_not(prev_tail)))
                def _w_full():
                    wait_scatter(tch)

                @pl.when(jnp.logical_and(do_wait, prev_tail))
                def _w_tail():
                    wait_scatter(ntail)

                def rot_body(t8, carry3):
                    for r8 in range(8):
                        row = t8 * 8 + r8
                        for w in range(ngrp):
                            buf_st[slot, row, pl.ds(16 * w, 16)] = buf_in[
                                _TAU * i + row, pl.ds(16 * w, 16)
                            ]
                    return carry3

                lax.fori_loop(0, tch // 8, rot_body, 0)

                ci = pl.multiple_of(i * C, 128)

                @pl.when(jnp.logical_not(is_tail))
                def _scatter_full():
                    pltpu.async_copy(
                        buf_st.at[slot, pl.ds(0, tch), :],
                        out_hbm.at[b, pl.ds(t0, tch), pl.ds(ci, C)],
                        ssem,
                    )

                @pl.when(is_tail)
                def _scatter_tail():
                    pltpu.async_copy(
                        buf_st.at[slot, pl.ds(0, ntail), :],
                        out_hbm.at[b, pl.ds(t0, ntail), pl.ds(ci, C)],
                        ssem,
                    )

                return carry2

            lax.fori_loop(0, nwin, win_body, 0)
            return carry

        lax.fori_loop(0, nch, chunk_body, 0)

        @pl.when(j < nblk - 1)
        def _drain_full():
            wait_scatter(tch)
            wait_scatter(tch)

        @pl.when(j == nblk - 1)
        def _drain_tail():
            wait_scatter(ntail)
            wait_scatter(ntail)

    return run(x)


# speedup vs baseline: 1.3555x; 1.3555x over previous
import functools

import jax
import jax.numpy as jnp
from jax import lax
from jax.experimental import pallas as pl
from jax.experimental.pallas import tpu as pltpu
from jax.experimental.pallas import tpu_sc as plsc

_M = 7
_TAU = 3


def kernel(x):
    B, S, C = x.shape
    nwin = _M + 1
    halo = _M * _TAU
    valid_t = S - halo
    nblk = 16
    tblk = 128
    tail = valid_t - (nblk - 1) * tblk
    tch = 32
    nch = tblk // tch
    glen = tch + ((halo + 7) // 8) * 8
    ntail = ((tail - (nch - 1) * tch + 7) // 8) * 8
    ngrp = C // 16

    mesh = plsc.VectorSubcoreMesh(core_axis_name="c", subcore_axis_name="s")

    @functools.partial(
        pl.kernel,
        mesh=mesh,
        out_type=jax.ShapeDtypeStruct((B, valid_t, nwin * C), jnp.float32),
        scratch_types=[
            pltpu.VMEM((glen, C), jnp.float32),
            pltpu.VMEM((2, tch, C), jnp.float32),
            pltpu.SemaphoreType.DMA,
            pltpu.SemaphoreType.DMA,
            pltpu.SemaphoreType.DMA,
        ],
        compiler_params=pltpu.CompilerParams(disable_bounds_checks=True),
    )
    def run(x_hbm, out_hbm, buf_in, buf_st, gsem, ssem, s0sem):
        cc = lax.axis_index("c")
        ss = lax.axis_index("s")
        wid = ss * 2 + cc
        b = wid // nblk
        j = wid % nblk

        def wait_scatter(sem, rows):
            pltpu.make_async_copy(
                buf_st.at[0, pl.ds(0, rows), :],
                out_hbm.at[b, pl.ds(0, rows), pl.ds(0, C)],
                sem,
            ).wait()

        def chunk_body(k, carry):
            t0 = pl.multiple_of(j * tblk + k * tch, 8)
            is_tail = jnp.logical_and(j == nblk - 1, k == nch - 1)

            @pl.when(k > 0)
            def _w0():
                wait_scatter(s0sem, tch)

            @pl.when(jnp.logical_not(is_tail))
            def _gather_full():
                pltpu.async_copy(
                    x_hbm.at[b, pl.ds(t0, glen), :], buf_in, gsem
                ).wait()

            @pl.when(is_tail)
            def _gather_tail():
                pltpu.async_copy(
                    x_hbm.at[b, pl.ds(t0, tch), :],
                    buf_in.at[pl.ds(0, tch)],
                    gsem,
                ).wait()

            @pl.when(jnp.logical_not(is_tail))
            def _scatter0_full():
                pltpu.async_copy(
                    buf_in.at[pl.ds(0, tch), :],
                    out_hbm.at[b, pl.ds(t0, tch), pl.ds(0, C)],
                    s0sem,
                )

            @pl.when(is_tail)
            def _scatter0_tail():
                pltpu.async_copy(
                    buf_in.at[pl.ds(0, ntail), :],
                    out_hbm.at[b, pl.ds(t0, ntail), pl.ds(0, C)],
                    s0sem,
                )

            def win_body(i, carry2):
                slot = lax.rem(i - 1, 2)
                do_wait = jnp.logical_or(k > 0, i >= 3)
                prev_tail = jnp.logical_and(is_tail, i >= 3)

                @pl.when(jnp.logical_and(do_wait, jnp.logical_not(prev_tail)))
                def _w_full():
                    wait_scatter(ssem, tch)

                @pl.when(jnp.logical_and(do_wait, prev_tail))
                def _w_tail():
                    wait_scatter(ssem, ntail)

                @plsc.parallel_loop(0, tch)
                def rot_body(row):
                    src = _TAU * i + row
                    vals = [
                        buf_in[src, pl.ds(16 * w, 16)] for w in range(ngrp)
                    ]
                    for w in range(ngrp):
                        buf_st[slot, row, pl.ds(16 * w, 16)] = vals[w]

                ci = pl.multiple_of(i * C, 128)

                @pl.when(jnp.logical_not(is_tail))
                def _scatter_full():
                    pltpu.async_copy(
                        buf_st.at[slot, pl.ds(0, tch), :],
                        out_hbm.at[b, pl.ds(t0, tch), pl.ds(ci, C)],
                        ssem,
                    )

                @pl.when(is_tail)
                def _scatter_tail():
                    pltpu.async_copy(
                        buf_st.at[slot, pl.ds(0, ntail), :],
                        out_hbm.at[b, pl.ds(t0, ntail), pl.ds(ci, C)],
                        ssem,
                    )

                return carry2

            lax.fori_loop(1, nwin, win_body, 0)
            return carry

        lax.fori_loop(0, nch, chunk_body, 0)

        @pl.when(j < nblk - 1)
        def _drain_full():
            wait_scatter(ssem, tch)
            wait_scatter(ssem, tch)
            wait_scatter(s0sem, tch)

        @pl.when(j == nblk - 1)
        def _drain_tail():
            wait_scatter(ssem, ntail)
            wait_scatter(ssem, ntail)
            wait_scatter(s0sem, ntail)

    return run(x)


# speedup vs baseline: 6.5309x; 4.8179x over previous
import functools

import jax
import jax.numpy as jnp
from jax import lax
from jax.experimental import pallas as pl
from jax.experimental.pallas import tpu as pltpu
from jax.experimental.pallas import tpu_sc as plsc

_M = 7
_TAU = 3


def kernel(x):
    B, S, C = x.shape
    nwin = _M + 1
    halo = _M * _TAU
    valid_t = S - halo
    nsc = 32
    ttile = 64
    tch = 24
    glen = 48
    nrow = B * nwin * (C // 128)
    ngr = nrow // 8
    last_t0 = ((valid_t - ttile - 1) // 8 + 1) * 8
    tail_l2 = valid_t - last_t0 - 2 * tch

    mesh = plsc.VectorSubcoreMesh(core_axis_name="c", subcore_axis_name="s")

    @functools.partial(
        pl.kernel,
        mesh=mesh,
        out_type=jax.ShapeDtypeStruct((valid_t, nrow, 128), jnp.float32),
        scratch_types=[
            pltpu.VMEM((B, glen, C), jnp.float32),
            pltpu.VMEM((2, tch, 8, 128), jnp.float32),
            pltpu.SemaphoreType.DMA,
            pltpu.SemaphoreType.DMA,
        ],
    )
    def run(x_hbm, out_hbm, buf_in, buf_st, gsem, ssem):
        cc_ = lax.axis_index("c")
        ss_ = lax.axis_index("s")
        wid = ss_ * 2 + cc_
        t0 = jnp.where(wid < nsc - 1, wid * ttile, last_t0)

        def wait_scatter(rows):
            pltpu.make_async_copy(
                buf_st.at[0, pl.ds(0, rows), :, :],
                out_hbm.at[pl.ds(0, rows), pl.ds(0, 8), :],
                ssem,
            ).wait()

        def emit_groups(t0c, lc, first_chunk, prev_rows):
            for g in range(ngr):
                slot = g % 2
                if first_chunk is None:
                    if g < 2:
                        wait_scatter(prev_rows)
                    else:
                        wait_scatter(lc)
                else:
                    if g < 2:
                        @pl.when(jnp.logical_not(first_chunk))
                        def _w():
                            wait_scatter(prev_rows)
                    else:
                        wait_scatter(lc)

                @plsc.parallel_loop(0, lc)
                def rot_body(toff):
                    for half in range(2):
                        vals = []
                        for r8 in range(8):
                            row = 8 * g + r8
                            bb, col = row % 2, row // 2
                            win, c6 = col // (C // 128), col % (C // 128)
                            for w in range(4):
                                vals.append(
                                    buf_in[
                                        bb,
                                        _TAU * win + toff,
                                        pl.ds(128 * c6 + 64 * half + 16 * w, 16),
                                    ]
                                )
                        idx = 0
                        for r8 in range(8):
                            for w in range(4):
                                buf_st[
                                    slot,
                                    toff,
                                    r8,
                                    pl.ds(64 * half + 16 * w, 16),
                                ] = vals[idx]
                                idx += 1

                pltpu.async_copy(
                    buf_st.at[slot, pl.ds(0, lc), :, :],
                    out_hbm.at[pl.ds(t0c, lc), pl.ds(8 * g, 8), :],
                    ssem,
                )

        def chunk_body(k, carry):
            t0c = pl.multiple_of(t0 + k * tch, 8)
            pltpu.async_copy(
                x_hbm.at[:, pl.ds(t0c, glen), :], buf_in, gsem
            ).wait()
            emit_groups(t0c, tch, k == 0, tch)
            return carry

        lax.fori_loop(0, 2, chunk_body, 0)

        t0c2 = pl.multiple_of(t0 + 2 * tch, 8)
        g2 = tail_l2 + halo
        pltpu.async_copy(
            x_hbm.at[:, pl.ds(t0c2, g2), :],
            buf_in.at[:, pl.ds(0, g2), :],
            gsem,
        ).wait()
        emit_groups(t0c2, tail_l2, None, tch)
        wait_scatter(tail_l2)
        wait_scatter(tail_l2)

    out3 = run(x)
    return (
        out3.reshape(valid_t, nwin * (C // 128), B, 128)
        .transpose(2, 0, 1, 3)
        .reshape(B, valid_t, nwin * C)
    )
